# Initial kernel scaffold; baseline (speedup 1.0000x reference)
import jax
import jax.numpy as jnp
from jax.experimental import pallas as pl


def kernel(x, edge_index, W_gcn0, b_gcn0, W_gcn1, b_gcn1, bn0_gamma, bn0_beta, bn1_gamma, bn1_beta, Wih0, Whh0, bih0, bhh0, Wih1, Whh1, bih1, bhh1, W_out, b_out):
    raise NotImplementedError("write your pallas kernel here")


# speedup vs baseline: 5.8060x; 5.8060x over previous
import functools

import jax
import jax.numpy as jnp
from jax import lax
from jax.experimental import pallas as pl

B = 32
T = 12
N = 325
F = 2
H = 128
E = 2600
O = 24
NB = B * N

_PREC = lax.Precision.HIGHEST


def _adj_body(edge_ref, a_ref):
    src = edge_ref[:, 0:1]
    dst = edge_ref[:, 1:2]
    col = lax.broadcasted_iota(jnp.int32, (E, N), 1)
    s_oh = (src == col).astype(jnp.float32)
    d_oh = (dst == col).astype(jnp.float32)
    deg = jnp.sum(d_oh, axis=0, keepdims=True) + 1.0
    dinv = lax.rsqrt(deg)
    dinv_s = lax.dot_general(s_oh, dinv, (((1,), (1,)), ((), ())),
                             precision=_PREC)
    dinv_d = lax.dot_general(d_oh, dinv, (((1,), (1,)), ((), ())),
                             precision=_PREC)
    sn = s_oh * (dinv_s * dinv_d)
    a = lax.dot_general(d_oh, sn, (((0,), (0,)), ((), ())),
                        precision=_PREC)
    ri = lax.broadcasted_iota(jnp.int32, (N, N), 0)
    ci = lax.broadcasted_iota(jnp.int32, (N, N), 1)
    a_ref[...] = a + jnp.where(ri == ci, dinv * dinv, 0.0)


def _build_adj(edge_index):
    edges = edge_index.T
    return pl.pallas_call(
        _adj_body,
        out_shape=jax.ShapeDtypeStruct((N, N), jnp.float32),
    )(edges)


def _bn_relu(h, gamma, beta, eps=1e-5):
    mu = jnp.mean(h, axis=(0, 1), keepdims=True)
    var = jnp.mean(h * h, axis=(0, 1), keepdims=True) - mu * mu
    out = (h - mu) * lax.rsqrt(var + eps) * gamma + beta
    return jnp.maximum(out, 0.0)


def _gcn_body(x_ref, a_ref, w0_ref, b0_ref, w1_ref, b1_ref,
              g0_ref, be0_ref, g1_ref, be1_ref, out_ref):
    a = a_ref[...]
    w0 = w0_ref[...]
    w1 = w1_ref[...]
    h0 = []
    for b in range(B):
        hb = jnp.dot(x_ref[0, b], w0, precision=_PREC)
        h0.append(jnp.dot(a, hb, precision=_PREC))
    h = jnp.stack(h0, axis=0) + b0_ref[...][None, None, :]
    h = _bn_relu(h, g0_ref[...][None, None, :], be0_ref[...][None, None, :])
    h1 = []
    for b in range(B):
        hb = jnp.dot(h[b], w1, precision=_PREC)
        h1.append(jnp.dot(a, hb, precision=_PREC))
    h = jnp.stack(h1, axis=0) + b1_ref[...][None, None, :]
    h = _bn_relu(h, g1_ref[...][None, None, :], be1_ref[...][None, None, :])
    out_ref[0] = h


def _gcn_stage(xs, a, w0, b0, w1, b1, g0, be0, g1, be1):
    full = lambda s: pl.BlockSpec(s, lambda t: (0,) * len(s))
    return pl.pallas_call(
        _gcn_body,
        grid=(T,),
        in_specs=[
            pl.BlockSpec((1, B, N, F), lambda t: (t, 0, 0, 0)),
            full((N, N)), full((F, H)), full((H,)), full((H, H)), full((H,)),
            full((H,)), full((H,)), full((H,)), full((H,)),
        ],
        out_specs=pl.BlockSpec((1, B, N, H), lambda t: (t, 0, 0, 0)),
        out_shape=jax.ShapeDtypeStruct((T, B, N, H), jnp.float32),
    )(xs, a, w0, b0, w1, b1, g0, be0, g1, be1)


_M = 1040


def _lstm_body(x_ref, wih0_ref, whh0_ref, b0_ref, wih1_ref, whh1_ref,
               b1_ref, wout_ref, bout_ref, out_ref):
    wih0 = wih0_ref[...]
    whh0 = whh0_ref[...]
    wih1 = wih1_ref[...]
    whh1 = whh1_ref[...]
    b0 = b0_ref[...][None, :]
    b1 = b1_ref[...][None, :]

    def gates(xt, h, wih, whh, bb):
        g = (jnp.dot(xt, wih, precision=_PREC)
             + jnp.dot(h, whh, precision=_PREC) + bb)
        i = jax.nn.sigmoid(g[:, 0 * H:1 * H])
        f = jax.nn.sigmoid(g[:, 1 * H:2 * H])
        gg = jnp.tanh(g[:, 2 * H:3 * H])
        o = jax.nn.sigmoid(g[:, 3 * H:4 * H])
        return i, f, gg, o

    z = jnp.zeros((_M, H), dtype=jnp.float32)
    h0, c0, h1, c1 = z, z, z, z
    for t in range(T):
        i, f, g, o = gates(x_ref[t], h0, wih0, whh0, b0)
        c0 = f * c0 + i * g
        h0 = o * jnp.tanh(c0)
        i, f, g, o = gates(h0, h1, wih1, whh1, b1)
        c1 = f * c1 + i * g
        h1 = o * jnp.tanh(c1)
    out_ref[...] = (jnp.dot(h1, wout_ref[...], precision=_PREC)
                    + bout_ref[...][None, :])


def _lstm_stage(seq, wih0t, whh0t, b0, wih1t, whh1t, b1, wout, bout):
    full = lambda s: pl.BlockSpec(s, lambda i: (0,) * len(s))
    return pl.pallas_call(
        _lstm_body,
        grid=(NB // _M,),
        in_specs=[
            pl.BlockSpec((T, _M, H), lambda i: (0, i, 0)),
            full((H, 4 * H)), full((H, 4 * H)), full((4 * H,)),
            full((H, 4 * H)), full((H, 4 * H)), full((4 * H,)),
            full((H, O)), full((O,)),
        ],
        out_specs=pl.BlockSpec((_M, O), lambda i: (i, 0)),
        out_shape=jax.ShapeDtypeStruct((NB, O), jnp.float32),
    )(seq, wih0t, whh0t, b0, wih1t, whh1t, b1, wout, bout)


@jax.jit
def kernel(x, edge_index, W_gcn0, b_gcn0, W_gcn1, b_gcn1, bn0_gamma,
           bn0_beta, bn1_gamma, bn1_beta, Wih0, Whh0, bih0, bhh0, Wih1,
           Whh1, bih1, bhh1, W_out, b_out):
    a = _build_adj(edge_index)
    xs = x.transpose(1, 0, 2, 3)
    g = _gcn_stage(xs, a, W_gcn0, b_gcn0, W_gcn1, b_gcn1,
                   bn0_gamma, bn0_beta, bn1_gamma, bn1_beta)
    seq = g.reshape(T, NB, H)
    pred = _lstm_stage(seq, Wih0.T, Whh0.T, bih0 + bhh0,
                       Wih1.T, Whh1.T, bih1 + bhh1, W_out, b_out)
    return pred.reshape(B, N, O)


# speedup vs baseline: 20.7939x; 3.5815x over previous
import functools

import jax
import jax.numpy as jnp
from jax import lax
from jax.experimental import pallas as pl

B = 32
T = 12
N = 325
F = 2
H = 128
E = 2600
O = 24
NB = B * N

_PREC = lax.Precision.DEFAULT


def _adj_body(edge_ref, a_ref):
    src = edge_ref[:, 0:1]
    dst = edge_ref[:, 1:2]
    col = lax.broadcasted_iota(jnp.int32, (E, N), 1)
    s_oh = (src == col).astype(jnp.float32)
    d_oh = (dst == col).astype(jnp.float32)
    deg = jnp.sum(d_oh, axis=0, keepdims=True) + 1.0
    dinv = lax.rsqrt(deg)
    dinv_s = lax.dot_general(s_oh, dinv, (((1,), (1,)), ((), ())),
                             precision=_PREC)
    dinv_d = lax.dot_general(d_oh, dinv, (((1,), (1,)), ((), ())),
                             precision=_PREC)
    sn = s_oh * (dinv_s * dinv_d)
    a = lax.dot_general(d_oh, sn, (((0,), (0,)), ((), ())),
                        precision=_PREC)
    ri = lax.broadcasted_iota(jnp.int32, (N, N), 0)
    ci = lax.broadcasted_iota(jnp.int32, (N, N), 1)
    a_ref[...] = a + jnp.where(ri == ci, dinv * dinv, 0.0)


def _build_adj(edge_index):
    edges = edge_index.T
    return pl.pallas_call(
        _adj_body,
        out_shape=jax.ShapeDtypeStruct((N, N), jnp.float32),
    )(edges)


def _bn_relu(h, gamma, beta, eps=1e-5):
    mu = jnp.mean(h, axis=(0, 1), keepdims=True)
    var = jnp.mean(h * h, axis=(0, 1), keepdims=True) - mu * mu
    out = (h - mu) * lax.rsqrt(var + eps) * gamma + beta
    return jnp.maximum(out, 0.0)


def _gcn_body(x_ref, a_ref, w0_ref, b0_ref, w1_ref, b1_ref,
              g0_ref, be0_ref, g1_ref, be1_ref, out_ref):
    a = a_ref[...]
    w0 = w0_ref[...]
    w1 = w1_ref[...]
    h0 = []
    for b in range(B):
        hb = jnp.dot(x_ref[0, b], w0, precision=_PREC)
        h0.append(jnp.dot(a, hb, precision=_PREC))
    h = jnp.stack(h0, axis=0) + b0_ref[...][None, None, :]
    h = _bn_relu(h, g0_ref[...][None, None, :], be0_ref[...][None, None, :])
    h1 = []
    for b in range(B):
        hb = jnp.dot(h[b], w1, precision=_PREC)
        h1.append(jnp.dot(a, hb, precision=_PREC))
    h = jnp.stack(h1, axis=0) + b1_ref[...][None, None, :]
    h = _bn_relu(h, g1_ref[...][None, None, :], be1_ref[...][None, None, :])
    out_ref[0] = h


def _gcn_stage(xs, a, w0, b0, w1, b1, g0, be0, g1, be1):
    full = lambda s: pl.BlockSpec(s, lambda t: (0,) * len(s))
    return pl.pallas_call(
        _gcn_body,
        grid=(T,),
        in_specs=[
            pl.BlockSpec((1, B, N, F), lambda t: (t, 0, 0, 0)),
            full((N, N)), full((F, H)), full((H,)), full((H, H)), full((H,)),
            full((H,)), full((H,)), full((H,)), full((H,)),
        ],
        out_specs=pl.BlockSpec((1, B, N, H), lambda t: (t, 0, 0, 0)),
        out_shape=jax.ShapeDtypeStruct((T, B, N, H), jnp.float32),
    )(xs, a, w0, b0, w1, b1, g0, be0, g1, be1)


_M = 1040


def _lstm_body(x_ref, wih0_ref, whh0_ref, b0_ref, wih1_ref, whh1_ref,
               b1_ref, wout_ref, bout_ref, out_ref):
    wih0 = wih0_ref[...]
    whh0 = whh0_ref[...]
    wih1 = wih1_ref[...]
    whh1 = whh1_ref[...]
    b0 = b0_ref[...][None, :]
    b1 = b1_ref[...][None, :]

    def gates(xt, h, wih, whh, bb):
        g = (jnp.dot(xt, wih, precision=_PREC)
             + jnp.dot(h, whh, precision=_PREC) + bb)
        i = jax.nn.sigmoid(g[:, 0 * H:1 * H])
        f = jax.nn.sigmoid(g[:, 1 * H:2 * H])
        gg = jnp.tanh(g[:, 2 * H:3 * H])
        o = jax.nn.sigmoid(g[:, 3 * H:4 * H])
        return i, f, gg, o

    z = jnp.zeros((_M, H), dtype=jnp.float32)
    h0, c0, h1, c1 = z, z, z, z
    for t in range(T):
        i, f, g, o = gates(x_ref[t], h0, wih0, whh0, b0)
        c0 = f * c0 + i * g
        h0 = o * jnp.tanh(c0)
        i, f, g, o = gates(h0, h1, wih1, whh1, b1)
        c1 = f * c1 + i * g
        h1 = o * jnp.tanh(c1)
    out_ref[...] = (jnp.dot(h1, wout_ref[...], precision=_PREC)
                    + bout_ref[...][None, :])


def _lstm_stage(seq, wih0t, whh0t, b0, wih1t, whh1t, b1, wout, bout):
    full = lambda s: pl.BlockSpec(s, lambda i: (0,) * len(s))
    return pl.pallas_call(
        _lstm_body,
        grid=(NB // _M,),
        in_specs=[
            pl.BlockSpec((T, _M, H), lambda i: (0, i, 0)),
            full((H, 4 * H)), full((H, 4 * H)), full((4 * H,)),
            full((H, 4 * H)), full((H, 4 * H)), full((4 * H,)),
            full((H, O)), full((O,)),
        ],
        out_specs=pl.BlockSpec((_M, O), lambda i: (i, 0)),
        out_shape=jax.ShapeDtypeStruct((NB, O), jnp.float32),
    )(seq, wih0t, whh0t, b0, wih1t, whh1t, b1, wout, bout)


@jax.jit
def kernel(x, edge_index, W_gcn0, b_gcn0, W_gcn1, b_gcn1, bn0_gamma,
           bn0_beta, bn1_gamma, bn1_beta, Wih0, Whh0, bih0, bhh0, Wih1,
           Whh1, bih1, bhh1, W_out, b_out):
    a = _build_adj(edge_index)
    xs = x.transpose(1, 0, 2, 3)
    g = _gcn_stage(xs, a, W_gcn0, b_gcn0, W_gcn1, b_gcn1,
                   bn0_gamma, bn0_beta, bn1_gamma, bn1_beta)
    seq = g.reshape(T, NB, H)
    pred = _lstm_stage(seq, Wih0.T, Whh0.T, bih0 + bhh0,
                       Wih1.T, Whh1.T, bih1 + bhh1, W_out, b_out)
    return pred.reshape(B, N, O)


# speedup vs baseline: 31.0969x; 1.4955x over previous
import functools

import jax
import jax.numpy as jnp
from jax import lax
from jax.experimental import pallas as pl

B = 32
T = 12
N = 325
F = 2
H = 128
E = 2600
O = 24
NB = B * N

_PREC = lax.Precision.DEFAULT


def _adj_body(edge_ref, a_ref):
    src = edge_ref[:, 0:1]
    dst = edge_ref[:, 1:2]
    col = lax.broadcasted_iota(jnp.int32, (E, N), 1)
    s_oh = (src == col).astype(jnp.float32)
    d_oh = (dst == col).astype(jnp.float32)
    deg = jnp.sum(d_oh, axis=0, keepdims=True) + 1.0
    dinv = lax.rsqrt(deg)
    dinv_s = lax.dot_general(s_oh, dinv, (((1,), (1,)), ((), ())),
                             precision=_PREC)
    dinv_d = lax.dot_general(d_oh, dinv, (((1,), (1,)), ((), ())),
                             precision=_PREC)
    sn = s_oh * (dinv_s * dinv_d)
    a = lax.dot_general(d_oh, sn, (((0,), (0,)), ((), ())),
                        precision=_PREC)
    ri = lax.broadcasted_iota(jnp.int32, (N, N), 0)
    ci = lax.broadcasted_iota(jnp.int32, (N, N), 1)
    a_ref[...] = a + jnp.where(ri == ci, dinv * dinv, 0.0)


def _build_adj(edge_index):
    edges = edge_index.T
    return pl.pallas_call(
        _adj_body,
        out_shape=jax.ShapeDtypeStruct((N, N), jnp.float32),
    )(edges)


def _bn_relu_wide(h, gamma, beta, eps=1e-5):
    cs = jnp.sum(h, axis=0, keepdims=True)
    cs2 = jnp.sum(h * h, axis=0, keepdims=True)
    s = sum(cs[:, b * H:(b + 1) * H] for b in range(B))
    s2 = sum(cs2[:, b * H:(b + 1) * H] for b in range(B))
    mu = s / NB
    var = s2 / NB - mu * mu
    scale = gamma[None, :] * lax.rsqrt(var + eps)
    shift = beta[None, :] - mu * scale
    scale_w = jnp.concatenate([scale] * B, axis=1)
    shift_w = jnp.concatenate([shift] * B, axis=1)
    return jnp.maximum(h * scale_w + shift_w, 0.0)


def _gcn_body(x_ref, a_ref, w0bd_ref, b0w_ref, w1_ref, b1w_ref,
              g0_ref, be0_ref, g1_ref, be1_ref, out_ref):
    a = a_ref[...]
    ax = jnp.dot(a, x_ref[0], precision=_PREC)
    h = jnp.dot(ax, w0bd_ref[...], precision=_PREC) + b0w_ref[...]
    h = _bn_relu_wide(h, g0_ref[...], be0_ref[...])
    ah = jnp.dot(a, h, precision=_PREC)
    w1 = w1_ref[...]
    h = jnp.concatenate(
        [jnp.dot(ah[:, b * H:(b + 1) * H], w1, precision=_PREC)
         for b in range(B)], axis=1) + b1w_ref[...]
    h = _bn_relu_wide(h, g1_ref[...], be1_ref[...])
    out_ref[0] = h


def _gcn_stage(xcat, a, w0bd, b0w, w1, b1w, g0, be0, g1, be1):
    full = lambda s: pl.BlockSpec(s, lambda t: (0,) * len(s))
    return pl.pallas_call(
        _gcn_body,
        grid=(T,),
        in_specs=[
            pl.BlockSpec((1, N, B * F), lambda t: (t, 0, 0)),
            full((N, N)), full((B * F, B * H)), full((1, B * H)),
            full((H, H)), full((1, B * H)),
            full((H,)), full((H,)), full((H,)), full((H,)),
        ],
        out_specs=pl.BlockSpec((1, N, B * H), lambda t: (t, 0, 0)),
        out_shape=jax.ShapeDtypeStruct((T, N, B * H), jnp.float32),
    )(xcat, a, w0bd, b0w, w1, b1w, g0, be0, g1, be1)


_M = 2080

def _gate_scale():
    col = lax.broadcasted_iota(jnp.int32, (1, 4 * H), 1)
    is_g = (col >= 2 * H) & (col < 3 * H)
    return jnp.where(is_g, 1.0, 0.5)


def _lstm_body(x_ref, wih0_ref, whh0_ref, b0_ref, wih1_ref, whh1_ref,
               b1_ref, wout_ref, bout_ref, out_ref):
    gs = _gate_scale()
    wih0 = wih0_ref[...] * gs
    whh0 = whh0_ref[...] * gs
    wih1 = wih1_ref[...] * gs
    whh1 = whh1_ref[...] * gs
    b0 = b0_ref[...][None, :] * gs
    b1 = b1_ref[...][None, :] * gs

    def gates(xt, h, wih, whh, bb):
        g = (jnp.dot(xt, wih, precision=_PREC)
             + jnp.dot(h, whh, precision=_PREC) + bb)
        th = jnp.tanh(g)
        i = 0.5 + 0.5 * th[:, 0 * H:1 * H]
        f = 0.5 + 0.5 * th[:, 1 * H:2 * H]
        gg = th[:, 2 * H:3 * H]
        o = 0.5 + 0.5 * th[:, 3 * H:4 * H]
        return i, f, gg, o

    z = jnp.zeros((_M, H), dtype=jnp.float32)
    h0, c0, h1, c1 = z, z, z, z
    for t in range(T):
        i, f, g, o = gates(x_ref[t], h0, wih0, whh0, b0)
        c0 = f * c0 + i * g
        h0 = o * jnp.tanh(c0)
        i, f, g, o = gates(h0, h1, wih1, whh1, b1)
        c1 = f * c1 + i * g
        h1 = o * jnp.tanh(c1)
    out_ref[...] = (jnp.dot(h1, wout_ref[...], precision=_PREC)
                    + bout_ref[...][None, :])


def _lstm_stage(seq, wih0t, whh0t, b0, wih1t, whh1t, b1, wout, bout):
    full = lambda s: pl.BlockSpec(s, lambda i: (0,) * len(s))
    return pl.pallas_call(
        _lstm_body,
        grid=(NB // _M,),
        in_specs=[
            pl.BlockSpec((T, _M, H), lambda i: (0, i, 0)),
            full((H, 4 * H)), full((H, 4 * H)), full((4 * H,)),
            full((H, 4 * H)), full((H, 4 * H)), full((4 * H,)),
            full((H, O)), full((O,)),
        ],
        out_specs=pl.BlockSpec((_M, O), lambda i: (i, 0)),
        out_shape=jax.ShapeDtypeStruct((NB, O), jnp.float32),
    )(seq, wih0t, whh0t, b0, wih1t, whh1t, b1, wout, bout)


@jax.jit
def kernel(x, edge_index, W_gcn0, b_gcn0, W_gcn1, b_gcn1, bn0_gamma,
           bn0_beta, bn1_gamma, bn1_beta, Wih0, Whh0, bih0, bhh0, Wih1,
           Whh1, bih1, bhh1, W_out, b_out):
    a = _build_adj(edge_index)
    xcat = x.transpose(1, 2, 0, 3).reshape(T, N, B * F)
    w0bd = jnp.kron(jnp.eye(B, dtype=jnp.float32), W_gcn0)
    b0w = jnp.tile(b_gcn0, B)[None, :]
    b1w = jnp.tile(b_gcn1, B)[None, :]
    g = _gcn_stage(xcat, a, w0bd, b0w, W_gcn1, b1w,
                   bn0_gamma, bn0_beta, bn1_gamma, bn1_beta)
    seq = g.reshape(T, NB, H)
    pred = _lstm_stage(seq, Wih0.T, Whh0.T, bih0 + bhh0,
                       Wih1.T, Whh1.T, bih1 + bhh1, W_out, b_out)
    return pred.reshape(N, B, O).transpose(1, 0, 2)
